# baseline (speedup 1.0000x reference)
import functools

import jax
import jax.numpy as jnp
from jax import lax
from jax.experimental import pallas as pl
from jax.experimental.pallas import tpu as pltpu
from jax.experimental.pallas import tpu_sc as plsc

N = 10000
E = 320000
L = 100000
D = 128

NC = 2
NS = 16
NTILES = NC * NS

CH = 128
N_PAD = 10240
RPT = N_PAD // NS
E_PT = E // NTILES
NCH_E = -(-E_PT // CH)
NCH_E = 80
E_PAD = NTILES * NCH_E * CH
NCH_L = 27
L_PT = NCH_L * CH
L_PAD = NTILES * L_PT
NBUF_E = 2
HALF_E = NCH_E // 2
NBUF_L = 3

BL = 1280
BLL = 2048


def _mesh():
  return plsc.VectorSubcoreMesh(core_axis_name="c", subcore_axis_name="s")


def _sc_degree(dst3):
  @functools.partial(
      pl.kernel,
      out_type=jax.ShapeDtypeStruct((NC, N_PAD), jnp.float32),
      mesh=_mesh(),
      scratch_types=[
          pltpu.VMEM((NCH_E, CH), jnp.int32),
          pltpu.VMEM((CH,), jnp.float32),
          pltpu.VMEM((RPT,), jnp.float32),
          pltpu.VMEM_SHARED((N_PAD,), jnp.float32),
          pltpu.SemaphoreType.DMA,
      ],
  )
  def k(dst_hbm, out_hbm, idx_v, ones_v, zeros_v, deg_sh, sem):
    del sem
    c = lax.axis_index("c")
    s = lax.axis_index("s")
    tb = c * NS + s
    for i in range(CH // 16):
      ones_v[pl.ds(i * 16, 16)] = jnp.ones((16,), jnp.float32)
    for i in range(RPT // 16):
      zeros_v[pl.ds(i * 16, 16)] = jnp.zeros((16,), jnp.float32)
    rs = pl.ds(s * RPT, RPT)
    pltpu.sync_copy(zeros_v, deg_sh.at[rs])
    pltpu.sync_copy(dst_hbm.at[tb], idx_v)
    plsc.subcore_barrier()

    def body(j, carry):
      pltpu.sync_copy(ones_v, deg_sh.at[idx_v.at[j]], add=True)
      return carry

    lax.fori_loop(0, NCH_E, body, 0)
    plsc.subcore_barrier()
    pltpu.sync_copy(deg_sh.at[rs], out_hbm.at[c].at[rs])

  return k(dst3)


def _sc_scatter(hs, src3, dst3, zeros2):
  @functools.partial(
      pl.kernel,
      out_type=jax.ShapeDtypeStruct((NC, N_PAD, D), jnp.float32),
      mesh=_mesh(),
      scratch_types=[
          pltpu.VMEM((HALF_E, CH), jnp.int32),
          pltpu.VMEM((HALF_E, CH), jnp.int32),
          [pltpu.VMEM((CH, D), jnp.float32) for _ in range(NBUF_E)],
          pltpu.VMEM_SHARED((N_PAD, D), jnp.float32),
          [pltpu.SemaphoreType.DMA for _ in range(NBUF_E)],
      ],
  )
  def k(hs_hbm, src_hbm, dst_hbm, z_hbm, out_hbm, sidx, didx, rows, acc_sh,
        sems):
    c = lax.axis_index("c")
    s = lax.axis_index("s")
    tb = c * NS + s
    rs = pl.ds(s * RPT, RPT)

    @pl.when(c == 0)
    def _():
      pltpu.sync_copy(hs_hbm.at[rs], acc_sh.at[rs])

    @pl.when(c != 0)
    def _():
      pltpu.sync_copy(z_hbm.at[rs], acc_sh.at[rs])

    plsc.subcore_barrier()

    for h in range(2):
      pltpu.sync_copy(src_hbm.at[tb].at[pl.ds(h * HALF_E, HALF_E)], sidx)
      pltpu.sync_copy(dst_hbm.at[tb].at[pl.ds(h * HALF_E, HALF_E)], didx)
      for i in range(NBUF_E):
        pltpu.async_copy(hs_hbm.at[sidx.at[i]], rows[i], sems[i])

      def body(t, carry):
        j = t * NBUF_E
        for i in range(NBUF_E):
          pltpu.make_async_copy(hs_hbm.at[pl.ds(0, CH)], rows[i],
                                sems[i]).wait()
          pltpu.sync_copy(rows[i], acc_sh.at[didx.at[j + i]], add=True)

          @pl.when(j + i + NBUF_E < HALF_E)
          def _():
            pltpu.async_copy(hs_hbm.at[sidx.at[j + i + NBUF_E]], rows[i],
                             sems[i])

        return carry

      lax.fori_loop(0, HALF_E // NBUF_E, body, 0)
    plsc.subcore_barrier()
    pltpu.sync_copy(acc_sh.at[rs], out_hbm.at[c].at[rs])

  return k(hs, src3, dst3, zeros2)


def _sc_gather(hsrc, hdst, sl3, dl3):
  @functools.partial(
      pl.kernel,
      out_type=jax.ShapeDtypeStruct((2, L_PAD, D), jnp.float32),
      mesh=_mesh(),
      scratch_types=[
          pltpu.VMEM((NCH_L, CH), jnp.int32),
          pltpu.VMEM((NCH_L, CH), jnp.int32),
          [pltpu.VMEM((CH, D), jnp.float32) for _ in range(NBUF_L)],
          [pltpu.VMEM((CH, D), jnp.float32) for _ in range(NBUF_L)],
          [pltpu.SemaphoreType.DMA for _ in range(NBUF_L)],
          [pltpu.SemaphoreType.DMA for _ in range(NBUF_L)],
      ],
  )
  def k(hs_hbm, hd_hbm, sl_hbm, dl_hbm, out_hbm, sidx, didx, rows_a, rows_b,
        gsem, wsem):
    c = lax.axis_index("c")
    s = lax.axis_index("s")
    tb = c * NS + s
    pltpu.sync_copy(sl_hbm.at[tb], sidx)
    pltpu.sync_copy(dl_hbm.at[tb], didx)
    base = tb * L_PT

    def fire_gathers(i, j):
      pltpu.async_copy(hs_hbm.at[sidx.at[j]], rows_a[i], gsem[i])
      pltpu.async_copy(hd_hbm.at[didx.at[j]], rows_b[i], gsem[i])

    def drain(i, bufs, sem):
      pltpu.make_async_copy(hs_hbm.at[pl.ds(0, CH)], bufs[i], sem[i]).wait()
      pltpu.make_async_copy(hs_hbm.at[pl.ds(0, CH)], bufs[i], sem[i]).wait()

    for i in range(NBUF_L):
      fire_gathers(i, i)

    def body(t, carry):
      j = t * NBUF_L
      for i in range(NBUF_L):
        drain(i, rows_a, gsem)
        o = pl.ds(base + (j + i) * CH, CH)
        pltpu.async_copy(rows_a[i], out_hbm.at[0].at[o], wsem[i])
        pltpu.async_copy(rows_b[i], out_hbm.at[1].at[o], wsem[i])

        @pl.when(j + i + NBUF_L < NCH_L)
        def _():
          drain(i, rows_a, wsem)
          fire_gathers(i, j + i + NBUF_L)

      return carry

    lax.fori_loop(0, NCH_L // NBUF_L, body, 0)
    for i in range(NBUF_L):
      drain(i, rows_a, wsem)

  return k(hsrc, hdst, sl3, dl3)


def _k_stage1(x_ref, w_ref, d0_ref, d1_ref, o_ref):
  dinv = lax.rsqrt(d0_ref[...] + d1_ref[...] + 1.0)
  h = jnp.dot(x_ref[...], w_ref[...], preferred_element_type=jnp.float32)
  o_ref[...] = h * dinv


def _k_stage3(a0_ref, a1_ref, d0_ref, d1_ref, b_ref, w_ref, o_ref):
  dinv = lax.rsqrt(d0_ref[...] + d1_ref[...] + 1.0)
  h1 = jnp.maximum((a0_ref[...] + a1_ref[...]) * dinv + b_ref[...], 0.0)
  h = jnp.dot(h1, w_ref[...], preferred_element_type=jnp.float32)
  o_ref[...] = h * dinv


def _k_stage5(a0_ref, a1_ref, d0_ref, d1_ref, b_ref, wa_ref, wb_ref, o1_ref,
              o2_ref):
  dinv = lax.rsqrt(d0_ref[...] + d1_ref[...] + 1.0)
  h2 = (a0_ref[...] + a1_ref[...]) * dinv + b_ref[...]
  o1_ref[...] = jnp.dot(h2, wa_ref[...], preferred_element_type=jnp.float32)
  o2_ref[...] = jnp.dot(h2, wb_ref[...], preferred_element_type=jnp.float32)


def _k_stage7(g0_ref, g1_ref, b_ref, w_ref, b2_ref, o_ref):
  p = jnp.maximum(g0_ref[...] + g1_ref[...] + b_ref[...], 0.0)
  r = jnp.sum(p * w_ref[...], axis=1) + b2_ref[0, 0]
  o_ref[...] = r.reshape(BLL, 1)


def _row_spec(bl):
  return pl.BlockSpec((bl, D), lambda i: (i, 0))


def _col_spec(bl):
  return pl.BlockSpec((bl, 1), lambda i: (i, 0))


def _full_spec(shape):
  return pl.BlockSpec(shape, lambda i: tuple(0 for _ in shape))


def kernel(x, edge_index, edge_label_index, W1, b1, W2, b2, Wp1, bp1, Wp2,
           bp2):
  f32 = jnp.float32
  i32 = jnp.int32

  src = edge_index[0]
  dst = edge_index[1]
  pe = E_PAD - E
  src3 = jnp.concatenate([src, jnp.zeros((pe,), i32)]).reshape(
      NTILES, NCH_E, CH)
  dst3 = jnp.concatenate([dst, jnp.full((pe,), N, i32)]).reshape(
      NTILES, NCH_E, CH)
  plab = L_PAD - L
  sl3 = jnp.concatenate([edge_label_index[0],
                         jnp.zeros((plab,), i32)]).reshape(NTILES, NCH_L, CH)
  dl3 = jnp.concatenate([edge_label_index[1],
                         jnp.zeros((plab,), i32)]).reshape(NTILES, NCH_L, CH)
  x_p = jnp.zeros((N_PAD, D), f32).at[:N].set(x)
  zeros2 = jnp.zeros((N_PAD, D), f32)

  deg = _sc_degree(dst3)
  d0 = deg[0].reshape(N_PAD, 1)
  d1 = deg[1].reshape(N_PAD, 1)

  grid_n = (N_PAD // BL,)

  hs1 = pl.pallas_call(
      _k_stage1,
      grid=grid_n,
      in_specs=[
          _row_spec(BL),
          _full_spec((D, D)),
          _col_spec(BL),
          _col_spec(BL),
      ],
      out_specs=_row_spec(BL),
      out_shape=jax.ShapeDtypeStruct((N_PAD, D), f32),
  )(x_p, W1, d0, d1)

  acc1 = _sc_scatter(hs1, src3, dst3, zeros2)

  hs2 = pl.pallas_call(
      _k_stage3,
      grid=grid_n,
      in_specs=[
          _row_spec(BL),
          _row_spec(BL),
          _col_spec(BL),
          _col_spec(BL),
          _full_spec((1, D)),
          _full_spec((D, D)),
      ],
      out_specs=_row_spec(BL),
      out_shape=jax.ShapeDtypeStruct((N_PAD, D), f32),
  )(acc1[0], acc1[1], d0, d1, b1.reshape(1, D), W2)

  acc2 = _sc_scatter(hs2, src3, dst3, zeros2)

  hsrc, hdst = pl.pallas_call(
      _k_stage5,
      grid=grid_n,
      in_specs=[
          _row_spec(BL),
          _row_spec(BL),
          _col_spec(BL),
          _col_spec(BL),
          _full_spec((1, D)),
          _full_spec((D, D)),
          _full_spec((D, D)),
      ],
      out_specs=[_row_spec(BL), _row_spec(BL)],
      out_shape=[
          jax.ShapeDtypeStruct((N_PAD, D), f32),
          jax.ShapeDtypeStruct((N_PAD, D), f32),
      ],
  )(acc2[0], acc2[1], d0, d1, b2.reshape(1, D), Wp1[:D], Wp1[D:])

  g = _sc_gather(hsrc, hdst, sl3, dl3)

  out = pl.pallas_call(
      _k_stage7,
      grid=(L_PAD // BLL,),
      in_specs=[
          _row_spec(BLL),
          _row_spec(BLL),
          _full_spec((1, D)),
          _full_spec((1, D)),
          _full_spec((1, 1)),
      ],
      out_specs=pl.BlockSpec((BLL, 1), lambda i: (i, 0)),
      out_shape=jax.ShapeDtypeStruct((L_PAD, 1), f32),
  )(g[0], g[1], bp1.reshape(1, D), Wp2.reshape(1, D), bp2.reshape(1, 1))

  return out.reshape(L_PAD)[:L]


# speedup vs baseline: 1.1145x; 1.1145x over previous
import functools

import jax
import jax.numpy as jnp
from jax import lax
from jax.experimental import pallas as pl
from jax.experimental.pallas import tpu as pltpu
from jax.experimental.pallas import tpu_sc as plsc

N = 10000
E = 320000
L = 100000
D = 128

NC = 2
NS = 16
NTILES = NC * NS

CH = 128
N_PAD = 10240
RPT = N_PAD // NS
E_PT = E // NTILES
NCH_E = -(-E_PT // CH)
NCH_E = 80
E_PAD = NTILES * NCH_E * CH
NCH_L = 26
L_PT = NCH_L * CH
L_PAD = NTILES * L_PT
NBUF_E = 2
HALF_E = NCH_E // 2
NBUF_L = 2

BL = 1280
BLL = 2048


def _mesh():
  return plsc.VectorSubcoreMesh(core_axis_name="c", subcore_axis_name="s")


def _sc_degree(dst3):
  @functools.partial(
      pl.kernel,
      out_type=jax.ShapeDtypeStruct((NC, N_PAD), jnp.float32),
      mesh=_mesh(),
      scratch_types=[
          pltpu.VMEM((NCH_E, CH), jnp.int32),
          pltpu.VMEM((CH,), jnp.float32),
          pltpu.VMEM((RPT,), jnp.float32),
          pltpu.VMEM_SHARED((N_PAD,), jnp.float32),
          pltpu.SemaphoreType.DMA,
      ],
  )
  def k(dst_hbm, out_hbm, idx_v, ones_v, zeros_v, deg_sh, sem):
    del sem
    c = lax.axis_index("c")
    s = lax.axis_index("s")
    tb = c * NS + s
    for i in range(CH // 16):
      ones_v[pl.ds(i * 16, 16)] = jnp.ones((16,), jnp.float32)
    for i in range(RPT // 16):
      zeros_v[pl.ds(i * 16, 16)] = jnp.zeros((16,), jnp.float32)
    rs = pl.ds(s * RPT, RPT)
    pltpu.sync_copy(zeros_v, deg_sh.at[rs])
    pltpu.sync_copy(dst_hbm.at[tb], idx_v)
    plsc.subcore_barrier()

    def body(j, carry):
      pltpu.sync_copy(ones_v, deg_sh.at[idx_v.at[j]], add=True)
      return carry

    lax.fori_loop(0, NCH_E, body, 0)
    plsc.subcore_barrier()
    pltpu.sync_copy(deg_sh.at[rs], out_hbm.at[c].at[rs])

  return k(dst3)


def _sc_scatter(hs, src3, dst3, zeros2):
  @functools.partial(
      pl.kernel,
      out_type=jax.ShapeDtypeStruct((NC, N_PAD, D), jnp.float32),
      mesh=_mesh(),
      scratch_types=[
          pltpu.VMEM((HALF_E, CH), jnp.int32),
          pltpu.VMEM((HALF_E, CH), jnp.int32),
          [pltpu.VMEM((CH, D), jnp.float32) for _ in range(NBUF_E)],
          pltpu.VMEM_SHARED((N_PAD, D), jnp.float32),
          [pltpu.SemaphoreType.DMA for _ in range(NBUF_E)],
      ],
  )
  def k(hs_hbm, src_hbm, dst_hbm, z_hbm, out_hbm, sidx, didx, rows, acc_sh,
        sems):
    c = lax.axis_index("c")
    s = lax.axis_index("s")
    tb = c * NS + s
    rs = pl.ds(s * RPT, RPT)

    @pl.when(c == 0)
    def _():
      pltpu.sync_copy(hs_hbm.at[rs], acc_sh.at[rs])

    @pl.when(c != 0)
    def _():
      pltpu.sync_copy(z_hbm.at[rs], acc_sh.at[rs])

    plsc.subcore_barrier()

    for h in range(2):
      pltpu.sync_copy(src_hbm.at[tb].at[pl.ds(h * HALF_E, HALF_E)], sidx)
      pltpu.sync_copy(dst_hbm.at[tb].at[pl.ds(h * HALF_E, HALF_E)], didx)
      for i in range(NBUF_E):
        pltpu.async_copy(hs_hbm.at[sidx.at[i]], rows[i], sems[i])

      def body(t, carry):
        j = t * NBUF_E
        for i in range(NBUF_E):
          pltpu.make_async_copy(hs_hbm.at[pl.ds(0, CH)], rows[i],
                                sems[i]).wait()
          pltpu.sync_copy(rows[i], acc_sh.at[didx.at[j + i]], add=True)

          @pl.when(j + i + NBUF_E < HALF_E)
          def _():
            pltpu.async_copy(hs_hbm.at[sidx.at[j + i + NBUF_E]], rows[i],
                             sems[i])

        return carry

      lax.fori_loop(0, HALF_E // NBUF_E, body, 0)
    plsc.subcore_barrier()
    pltpu.sync_copy(acc_sh.at[rs], out_hbm.at[c].at[rs])

  return k(hs, src3, dst3, zeros2)


def _sc_gather_predict(hsrc, hdst, sl3, dl3, bvec, wvec):
  @functools.partial(
      pl.kernel,
      out_type=jax.ShapeDtypeStruct((L_PAD, 16), jnp.float32),
      mesh=_mesh(),
      scratch_types=[
          pltpu.VMEM((NCH_L, CH), jnp.int32),
          pltpu.VMEM((NCH_L, CH), jnp.int32),
          [pltpu.VMEM((CH, D), jnp.float32) for _ in range(NBUF_L)],
          [pltpu.VMEM((CH, D), jnp.float32) for _ in range(NBUF_L)],
          [pltpu.VMEM((CH, 16), jnp.float32) for _ in range(NBUF_L)],
          pltpu.VMEM((D,), jnp.float32),
          pltpu.VMEM((D,), jnp.float32),
          [pltpu.SemaphoreType.DMA for _ in range(NBUF_L)],
          [pltpu.SemaphoreType.DMA for _ in range(NBUF_L)],
      ],
  )
  def k(hs_hbm, hd_hbm, sl_hbm, dl_hbm, b_hbm, w_hbm, out_hbm, sidx, didx,
        rows_a, rows_b, pbuf, b_v, w_v, gsem, wsem):
    c = lax.axis_index("c")
    s = lax.axis_index("s")
    tb = c * NS + s
    pltpu.sync_copy(sl_hbm.at[tb], sidx)
    pltpu.sync_copy(dl_hbm.at[tb], didx)
    pltpu.sync_copy(b_hbm, b_v)
    pltpu.sync_copy(w_hbm, w_v)
    base = tb * L_PT
    bp = [b_v[pl.ds(w * 16, 16)] for w in range(D // 16)]
    wp = [w_v[pl.ds(w * 16, 16)] for w in range(D // 16)]

    def fire_gathers(i, j):
      pltpu.async_copy(hs_hbm.at[sidx.at[j]], rows_a[i], gsem[i])
      pltpu.async_copy(hd_hbm.at[didx.at[j]], rows_b[i], gsem[i])

    def drain_g(i):
      pltpu.make_async_copy(hs_hbm.at[pl.ds(0, CH)], rows_a[i],
                            gsem[i]).wait()
      pltpu.make_async_copy(hs_hbm.at[pl.ds(0, CH)], rows_a[i],
                            gsem[i]).wait()

    def drain_w(i):
      pltpu.make_async_copy(out_hbm.at[pl.ds(0, CH)], pbuf[i],
                            wsem[i]).wait()

    for i in range(NBUF_L):
      fire_gathers(i, i)

    def body(t, carry):
      j = t * NBUF_L
      for i in range(NBUF_L):
        drain_g(i)

        def row(r, rcarry):
          acc = jnp.zeros((16,), jnp.float32)
          for w in range(D // 16):
            a = rows_a[i][r, pl.ds(w * 16, 16)]
            b = rows_b[i][r, pl.ds(w * 16, 16)]
            acc = acc + jnp.maximum(a + b + bp[w], 0.0) * wp[w]
          pbuf[i][r] = acc
          return rcarry

        lax.fori_loop(0, CH, row, 0)
        pltpu.async_copy(pbuf[i], out_hbm.at[pl.ds(base + (j + i) * CH, CH)],
                         wsem[i])

        @pl.when(j + i + NBUF_L < NCH_L)
        def _():
          drain_w(i)
          fire_gathers(i, j + i + NBUF_L)

      return carry

    lax.fori_loop(0, NCH_L // NBUF_L, body, 0)
    for i in range(NBUF_L):
      drain_w(i)

  return k(hsrc, hdst, sl3, dl3, bvec, wvec)


def _k_stage1(x_ref, w_ref, d0_ref, d1_ref, o_ref):
  dinv = lax.rsqrt(d0_ref[...] + d1_ref[...] + 1.0)
  h = jnp.dot(x_ref[...], w_ref[...], preferred_element_type=jnp.float32)
  o_ref[...] = h * dinv


def _k_stage3(a0_ref, a1_ref, d0_ref, d1_ref, b_ref, w_ref, o_ref):
  dinv = lax.rsqrt(d0_ref[...] + d1_ref[...] + 1.0)
  h1 = jnp.maximum((a0_ref[...] + a1_ref[...]) * dinv + b_ref[...], 0.0)
  h = jnp.dot(h1, w_ref[...], preferred_element_type=jnp.float32)
  o_ref[...] = h * dinv


def _k_stage5(a0_ref, a1_ref, d0_ref, d1_ref, b_ref, wa_ref, wb_ref, o1_ref,
              o2_ref):
  dinv = lax.rsqrt(d0_ref[...] + d1_ref[...] + 1.0)
  h2 = (a0_ref[...] + a1_ref[...]) * dinv + b_ref[...]
  o1_ref[...] = jnp.dot(h2, wa_ref[...], preferred_element_type=jnp.float32)
  o2_ref[...] = jnp.dot(h2, wb_ref[...], preferred_element_type=jnp.float32)


def _k_stage7(p_ref, b2_ref, o_ref):
  r = jnp.sum(p_ref[...], axis=1) + b2_ref[0, 0]
  o_ref[...] = r.reshape(BLL, 1)


def _row_spec(bl):
  return pl.BlockSpec((bl, D), lambda i: (i, 0))


def _col_spec(bl):
  return pl.BlockSpec((bl, 1), lambda i: (i, 0))


def _full_spec(shape):
  return pl.BlockSpec(shape, lambda i: tuple(0 for _ in shape))


def kernel(x, edge_index, edge_label_index, W1, b1, W2, b2, Wp1, bp1, Wp2,
           bp2):
  f32 = jnp.float32
  i32 = jnp.int32

  src = edge_index[0]
  dst = edge_index[1]
  pe = E_PAD - E
  src3 = jnp.concatenate([src, jnp.zeros((pe,), i32)]).reshape(
      NTILES, NCH_E, CH)
  dst3 = jnp.concatenate([dst, jnp.full((pe,), N, i32)]).reshape(
      NTILES, NCH_E, CH)
  plab = L_PAD - L
  sl3 = jnp.concatenate([edge_label_index[0],
                         jnp.zeros((plab,), i32)]).reshape(NTILES, NCH_L, CH)
  dl3 = jnp.concatenate([edge_label_index[1],
                         jnp.zeros((plab,), i32)]).reshape(NTILES, NCH_L, CH)
  x_p = jnp.zeros((N_PAD, D), f32).at[:N].set(x)
  zeros2 = jnp.zeros((N_PAD, D), f32)

  deg = _sc_degree(dst3)
  d0 = deg[0].reshape(N_PAD, 1)
  d1 = deg[1].reshape(N_PAD, 1)

  grid_n = (N_PAD // BL,)

  hs1 = pl.pallas_call(
      _k_stage1,
      grid=grid_n,
      in_specs=[
          _row_spec(BL),
          _full_spec((D, D)),
          _col_spec(BL),
          _col_spec(BL),
      ],
      out_specs=_row_spec(BL),
      out_shape=jax.ShapeDtypeStruct((N_PAD, D), f32),
  )(x_p, W1, d0, d1)

  acc1 = _sc_scatter(hs1, src3, dst3, zeros2)

  hs2 = pl.pallas_call(
      _k_stage3,
      grid=grid_n,
      in_specs=[
          _row_spec(BL),
          _row_spec(BL),
          _col_spec(BL),
          _col_spec(BL),
          _full_spec((1, D)),
          _full_spec((D, D)),
      ],
      out_specs=_row_spec(BL),
      out_shape=jax.ShapeDtypeStruct((N_PAD, D), f32),
  )(acc1[0], acc1[1], d0, d1, b1.reshape(1, D), W2)

  acc2 = _sc_scatter(hs2, src3, dst3, zeros2)

  hsrc, hdst = pl.pallas_call(
      _k_stage5,
      grid=grid_n,
      in_specs=[
          _row_spec(BL),
          _row_spec(BL),
          _col_spec(BL),
          _col_spec(BL),
          _full_spec((1, D)),
          _full_spec((D, D)),
          _full_spec((D, D)),
      ],
      out_specs=[_row_spec(BL), _row_spec(BL)],
      out_shape=[
          jax.ShapeDtypeStruct((N_PAD, D), f32),
          jax.ShapeDtypeStruct((N_PAD, D), f32),
      ],
  )(acc2[0], acc2[1], d0, d1, b2.reshape(1, D), Wp1[:D], Wp1[D:])

  p16 = _sc_gather_predict(hsrc, hdst, sl3, dl3, bp1, Wp2.reshape(D))

  out = pl.pallas_call(
      _k_stage7,
      grid=(L_PAD // BLL,),
      in_specs=[
          pl.BlockSpec((BLL, 16), lambda i: (i, 0)),
          _full_spec((1, 1)),
      ],
      out_specs=pl.BlockSpec((BLL, 1), lambda i: (i, 0)),
      out_shape=jax.ShapeDtypeStruct((L_PAD, 1), f32),
  )(p16, bp2.reshape(1, 1))

  return out.reshape(L_PAD)[:L]
